# baseline (speedup 1.0000x reference)
import functools

import jax
import jax.numpy as jnp
from jax import lax
from jax.experimental import pallas as pl
from jax.experimental.pallas import tpu as pltpu
from jax.experimental.pallas import tpu_sc as plsc

DIM = 64
PACK = 2 * DIM
E_ROWS = 1000000
E_MAIN = 999936
N_TAIL = E_ROWS - E_MAIN
BATCH = 16384
MARGIN = 1.0

NUM_CORES = 2
NUM_SUBCORES = 16
NUM_WORKERS = NUM_CORES * NUM_SUBCORES
ROWS_PER_W = BATCH // NUM_WORKERS
CHUNK = 128
CHUNKS_PER_HALF = ROWS_PER_W // CHUNK
GROUPS = CHUNK // 16


def _sc_body(idx_hbm, par_hbm, trow_hbm, e_hbm, r_hbm, et_hbm, rt_hbm,
             loss_hbm, pos_hbm, neg_hbm,
             idx_v, par_v, trow_v, etail_v, rtail_v,
             hbuf, rbuf, tbuf, dist_v, loss_v, sem):
    wid = lax.axis_index("s") * NUM_CORES + lax.axis_index("c")
    base = wid * ROWS_PER_W

    sl = pl.ds(wid * CHUNKS_PER_HALF, CHUNKS_PER_HALF)
    pltpu.sync_copy(idx_hbm.at[sl], idx_v)
    pltpu.sync_copy(par_hbm.at[sl], par_v)
    pltpu.sync_copy(trow_hbm.at[sl], trow_v)
    pltpu.sync_copy(et_hbm, etail_v)
    pltpu.sync_copy(rt_hbm, rtail_v)

    lane = lax.iota(jnp.int32, 16)

    def chunk_body(c, _):
        cc = c % CHUNKS_PER_HALF
        kind0 = (c // CHUNKS_PER_HALF) * 3

        dh = pltpu.async_copy(e_hbm.at[idx_v.at[cc, kind0]], hbuf, sem)
        dr = pltpu.async_copy(r_hbm.at[idx_v.at[cc, kind0 + 1]], rbuf, sem)
        dt = pltpu.async_copy(e_hbm.at[idx_v.at[cc, kind0 + 2]], tbuf, sem)
        dh.wait()
        dr.wait()
        dt.wait()

        def group_body(g, _):
            rows = g * 16 + lane
            gsl = pl.ds(g * 16, 16)
            hoff = par_v[cc, kind0, gsl]
            roff = par_v[cc, kind0 + 1, gsl]
            toff = par_v[cc, kind0 + 2, gsl]
            htr = trow_v[cc, kind0, gsl]
            rtr = trow_v[cc, kind0 + 1, gsl]
            ttr = trow_v[cc, kind0 + 2, gsl]
            hmask = htr >= 0
            rmask = rtr >= 0
            tmask = ttr >= 0
            hrow = jnp.maximum(htr, 0)
            rrow = jnp.maximum(rtr, 0)
            trow = jnp.maximum(ttr, 0)

            def ld(kind_buf, tail_buf, mask, trow_c, off, d):
                col = off + ((lane + d) & (DIM - 1))
                main = plsc.load_gather(kind_buf, [rows, col])
                tail = plsc.load_gather(tail_buf, [trow_c, col])
                return jnp.where(mask, tail, main)

            nh = jnp.zeros((16,), jnp.float32)
            nt = jnp.zeros((16,), jnp.float32)
            for d in range(DIM):
                nh = nh + jnp.abs(ld(hbuf, etail_v, hmask, hrow, hoff, d))
                nt = nt + jnp.abs(ld(tbuf, etail_v, tmask, trow, toff, d))
            inv_nh = 1.0 / nh
            inv_nt = 1.0 / nt
            acc = jnp.zeros((16,), jnp.float32)
            for d in range(DIM):
                hv = ld(hbuf, etail_v, hmask, hrow, hoff, d)
                rv = ld(rbuf, rtail_v, rmask, rrow, roff, d)
                tv = ld(tbuf, etail_v, tmask, trow, toff, d)
                acc = acc + jnp.abs(hv * inv_nh + rv - tv * inv_nt)
            dist_v[pl.ds(c * CHUNK + g * 16, 16)] = acc
            return 0

        lax.fori_loop(0, GROUPS, group_body, 0)
        return 0

    lax.fori_loop(0, 2 * CHUNKS_PER_HALF, chunk_body, 0)

    for v in range(ROWS_PER_W // 16):
        pv = dist_v[pl.ds(v * 16, 16)]
        nv = dist_v[pl.ds(ROWS_PER_W + v * 16, 16)]
        loss_v[pl.ds(v * 16, 16)] = jnp.maximum(pv - nv + MARGIN, 0.0)

    pltpu.sync_copy(loss_v, loss_hbm.at[pl.ds(base, ROWS_PER_W)])
    pltpu.sync_copy(dist_v.at[pl.ds(0, ROWS_PER_W)],
                    pos_hbm.at[pl.ds(base, ROWS_PER_W)])
    pltpu.sync_copy(dist_v.at[pl.ds(ROWS_PER_W, ROWS_PER_W)],
                    neg_hbm.at[pl.ds(base, ROWS_PER_W)])


_transe_sc = functools.partial(
    pl.kernel,
    out_type=(
        jax.ShapeDtypeStruct((BATCH,), jnp.float32),
        jax.ShapeDtypeStruct((BATCH,), jnp.float32),
        jax.ShapeDtypeStruct((BATCH,), jnp.float32),
    ),
    mesh=plsc.VectorSubcoreMesh(core_axis_name="c", subcore_axis_name="s",
                                num_cores=NUM_CORES,
                                num_subcores=NUM_SUBCORES),
    scratch_types=[
        pltpu.VMEM((CHUNKS_PER_HALF, 6, CHUNK), jnp.int32),
        pltpu.VMEM((CHUNKS_PER_HALF, 6, CHUNK), jnp.int32),
        pltpu.VMEM((CHUNKS_PER_HALF, 6, CHUNK), jnp.int32),
        pltpu.VMEM((N_TAIL // 2, PACK), jnp.float32),
        pltpu.VMEM((N_TAIL // 2, PACK), jnp.float32),
        pltpu.VMEM((CHUNK, PACK), jnp.float32),
        pltpu.VMEM((CHUNK, PACK), jnp.float32),
        pltpu.VMEM((CHUNK, PACK), jnp.float32),
        pltpu.VMEM((2 * ROWS_PER_W,), jnp.float32),
        pltpu.VMEM((ROWS_PER_W,), jnp.float32),
        pltpu.SemaphoreType.DMA,
    ],
    compiler_params=pltpu.CompilerParams(needs_layout_passes=False,
                                         use_tc_tiling_on_sc=False),
)(_sc_body)


TR_COLS = 32256
TR_GRID = E_MAIN // TR_COLS
NQ = 4
BAND = DIM // NQ


def _transpose_body(t_hbm, o_ref, buf, sem):
    i = pl.program_id(0)

    def issue(j, slot):
        for q in range(NQ):
            pltpu.make_async_copy(
                t_hbm.at[pl.ds(q * BAND, BAND), pl.ds(j * TR_COLS, TR_COLS)],
                buf.at[slot, pl.ds(q * BAND, BAND), :],
                sem.at[slot, q]).start()

    @pl.when(i == 0)
    def _():
        issue(0, 0)

    @pl.when(i + 1 < TR_GRID)
    def _():
        issue(i + 1, (i + 1) % 2)

    slot = i % 2
    for q in range(NQ):
        pltpu.make_async_copy(
            t_hbm.at[pl.ds(q * BAND, BAND), pl.ds(i * TR_COLS, TR_COLS)],
            buf.at[slot, pl.ds(q * BAND, BAND), :],
            sem.at[slot, q]).wait()

    ii = lax.broadcasted_iota(jnp.int32, (DIM, DIM), 0)
    jj = lax.broadcasted_iota(jnp.int32, (DIM, DIM), 1)
    ident = jnp.where(ii == jj, 1.0, 0.0).astype(jnp.float32)
    dn = (((0,), (0,)), ((), ()))
    o_ref[...] = lax.dot_general(buf[slot], ident, dn,
                                 preferred_element_type=jnp.float32)


def _relayout_one(table):
    out = pl.pallas_call(
        _transpose_body,
        grid=(TR_GRID,),
        in_specs=[pl.BlockSpec(memory_space=pltpu.MemorySpace.HBM)],
        out_specs=pl.BlockSpec((TR_COLS, DIM), lambda i: (i, 0)),
        out_shape=jax.ShapeDtypeStruct((E_MAIN, DIM), jnp.float32),
        scratch_shapes=[
            pltpu.VMEM((2, DIM, TR_COLS), jnp.float32),
            pltpu.SemaphoreType.DMA((2, NQ)),
        ],
    )(table.T)
    return out.reshape(-1, PACK)


def _pack_idx(pos_triplets, neg_triplets):
    p = pos_triplets.astype(jnp.int32).reshape(BATCH // CHUNK, CHUNK, 3)
    n = neg_triplets.astype(jnp.int32).reshape(BATCH // CHUNK, CHUNK, 3)
    tri = jnp.concatenate([p.transpose(0, 2, 1), n.transpose(0, 2, 1)],
                          axis=1)
    pidx = jnp.minimum(tri, E_MAIN - 1) >> 1
    par = (tri & 1) * DIM
    trow = (tri - E_MAIN) >> 1
    return pidx, par, trow


def kernel(pos_triplets, neg_triplets, e_table, r_table):
    pidx, par, trow = _pack_idx(pos_triplets, neg_triplets)
    e_packed = _relayout_one(e_table)
    r_packed = _relayout_one(r_table)
    e_tail = e_table[E_MAIN:E_ROWS].reshape(N_TAIL // 2, PACK)
    r_tail = r_table[E_MAIN:E_ROWS].reshape(N_TAIL // 2, PACK)
    loss, pos_d, neg_d = _transe_sc(pidx, par, trow, e_packed, r_packed,
                                    e_tail, r_tail)
    return (loss, pos_d, neg_d)


# speedup vs baseline: 1.0533x; 1.0533x over previous
import functools

import jax
import jax.numpy as jnp
from jax import lax
from jax.experimental import pallas as pl
from jax.experimental.pallas import tpu as pltpu
from jax.experimental.pallas import tpu_sc as plsc

DIM = 64
PACK = 2 * DIM
E_ROWS = 1000000
E_MAIN = 999936
N_TAIL = E_ROWS - E_MAIN
BATCH = 16384
MARGIN = 1.0

NUM_CORES = 2
NUM_SUBCORES = 16
NUM_WORKERS = NUM_CORES * NUM_SUBCORES
ROWS_PER_W = BATCH // NUM_WORKERS
CHUNK = 128
CHUNKS_PER_HALF = ROWS_PER_W // CHUNK
GROUPS = CHUNK // 16


def _sc_body(idx_hbm, par_hbm, trow_hbm, e_hbm, r_hbm, et_hbm, rt_hbm,
             loss_hbm, pos_hbm, neg_hbm,
             idx_v, par_v, trow_v, etail_v, rtail_v,
             hbuf, rbuf, tbuf, dist_v, loss_v, sem):
    wid = lax.axis_index("s") * NUM_CORES + lax.axis_index("c")
    base = wid * ROWS_PER_W

    sl = pl.ds(wid * CHUNKS_PER_HALF, CHUNKS_PER_HALF)
    pltpu.sync_copy(idx_hbm.at[sl], idx_v)
    pltpu.sync_copy(par_hbm.at[sl], par_v)
    pltpu.sync_copy(trow_hbm.at[sl], trow_v)
    pltpu.sync_copy(et_hbm, etail_v)
    pltpu.sync_copy(rt_hbm, rtail_v)

    lane = lax.iota(jnp.int32, 16)

    def chunk_body(c, _):
        cc = c % CHUNKS_PER_HALF
        kind0 = (c // CHUNKS_PER_HALF) * 3

        dh = pltpu.async_copy(e_hbm.at[idx_v.at[cc, kind0]], hbuf, sem)
        dr = pltpu.async_copy(r_hbm.at[idx_v.at[cc, kind0 + 1]], rbuf, sem)
        dt = pltpu.async_copy(e_hbm.at[idx_v.at[cc, kind0 + 2]], tbuf, sem)
        dh.wait()
        dr.wait()
        dt.wait()

        def group_body(g, _):
            rows = g * 16 + lane
            gsl = pl.ds(g * 16, 16)
            hoff = par_v[cc, kind0, gsl]
            roff = par_v[cc, kind0 + 1, gsl]
            toff = par_v[cc, kind0 + 2, gsl]
            htr = trow_v[cc, kind0, gsl]
            rtr = trow_v[cc, kind0 + 1, gsl]
            ttr = trow_v[cc, kind0 + 2, gsl]
            hmask = htr >= 0
            rmask = rtr >= 0
            tmask = ttr >= 0
            hrow = jnp.maximum(htr, 0)
            rrow = jnp.maximum(rtr, 0)
            trow = jnp.maximum(ttr, 0)

            def ld(kind_buf, tail_buf, mask, trow_c, off, d):
                col = off + ((lane + d) & (DIM - 1))
                main = plsc.load_gather(kind_buf, [rows, col])
                tail = plsc.load_gather(tail_buf, [trow_c, col])
                return jnp.where(mask, tail, main)

            nh = jnp.zeros((16,), jnp.float32)
            nt = jnp.zeros((16,), jnp.float32)
            for d in range(DIM):
                nh = nh + jnp.abs(ld(hbuf, etail_v, hmask, hrow, hoff, d))
                nt = nt + jnp.abs(ld(tbuf, etail_v, tmask, trow, toff, d))
            inv_nh = 1.0 / nh
            inv_nt = 1.0 / nt
            acc = jnp.zeros((16,), jnp.float32)
            for d in range(DIM):
                hv = ld(hbuf, etail_v, hmask, hrow, hoff, d)
                rv = ld(rbuf, rtail_v, rmask, rrow, roff, d)
                tv = ld(tbuf, etail_v, tmask, trow, toff, d)
                acc = acc + jnp.abs(hv * inv_nh + rv - tv * inv_nt)
            dist_v[pl.ds(c * CHUNK + g * 16, 16)] = acc
            return 0

        lax.fori_loop(0, GROUPS, group_body, 0)
        return 0

    lax.fori_loop(0, 2 * CHUNKS_PER_HALF, chunk_body, 0)

    for v in range(ROWS_PER_W // 16):
        pv = dist_v[pl.ds(v * 16, 16)]
        nv = dist_v[pl.ds(ROWS_PER_W + v * 16, 16)]
        loss_v[pl.ds(v * 16, 16)] = jnp.maximum(pv - nv + MARGIN, 0.0)

    pltpu.sync_copy(loss_v, loss_hbm.at[pl.ds(base, ROWS_PER_W)])
    pltpu.sync_copy(dist_v.at[pl.ds(0, ROWS_PER_W)],
                    pos_hbm.at[pl.ds(base, ROWS_PER_W)])
    pltpu.sync_copy(dist_v.at[pl.ds(ROWS_PER_W, ROWS_PER_W)],
                    neg_hbm.at[pl.ds(base, ROWS_PER_W)])


_transe_sc = functools.partial(
    pl.kernel,
    out_type=(
        jax.ShapeDtypeStruct((BATCH,), jnp.float32),
        jax.ShapeDtypeStruct((BATCH,), jnp.float32),
        jax.ShapeDtypeStruct((BATCH,), jnp.float32),
    ),
    mesh=plsc.VectorSubcoreMesh(core_axis_name="c", subcore_axis_name="s",
                                num_cores=NUM_CORES,
                                num_subcores=NUM_SUBCORES),
    scratch_types=[
        pltpu.VMEM((CHUNKS_PER_HALF, 6, CHUNK), jnp.int32),
        pltpu.VMEM((CHUNKS_PER_HALF, 6, CHUNK), jnp.int32),
        pltpu.VMEM((CHUNKS_PER_HALF, 6, CHUNK), jnp.int32),
        pltpu.VMEM((N_TAIL // 2, PACK), jnp.float32),
        pltpu.VMEM((N_TAIL // 2, PACK), jnp.float32),
        pltpu.VMEM((CHUNK, PACK), jnp.float32),
        pltpu.VMEM((CHUNK, PACK), jnp.float32),
        pltpu.VMEM((CHUNK, PACK), jnp.float32),
        pltpu.VMEM((2 * ROWS_PER_W,), jnp.float32),
        pltpu.VMEM((ROWS_PER_W,), jnp.float32),
        pltpu.SemaphoreType.DMA,
    ],
    compiler_params=pltpu.CompilerParams(needs_layout_passes=False,
                                         use_tc_tiling_on_sc=False),
)(_sc_body)


TR_COLS = 32256
TR_GRID = E_MAIN // TR_COLS
NQ = 4
BAND = DIM // NQ


def _transpose_body(t_hbm, o_ref, buf, sem):
    i = pl.program_id(0)

    def issue(j, slot):
        for q in range(NQ):
            pltpu.make_async_copy(
                t_hbm.at[pl.ds(q * BAND, BAND), pl.ds(j * TR_COLS, TR_COLS)],
                buf.at[slot, pl.ds(q * BAND, BAND), :],
                sem.at[slot, q]).start()

    @pl.when(i == 0)
    def _():
        issue(0, 0)

    @pl.when(i + 1 < TR_GRID)
    def _():
        issue(i + 1, (i + 1) % 2)

    slot = i % 2
    for q in range(NQ):
        pltpu.make_async_copy(
            t_hbm.at[pl.ds(q * BAND, BAND), pl.ds(i * TR_COLS, TR_COLS)],
            buf.at[slot, pl.ds(q * BAND, BAND), :],
            sem.at[slot, q]).wait()

    ii = lax.broadcasted_iota(jnp.int32, (DIM, DIM), 0)
    jj = lax.broadcasted_iota(jnp.int32, (DIM, DIM), 1)
    ident = jnp.where(ii == jj, 1.0, 0.0).astype(jnp.float32)
    dn = (((0,), (0,)), ((), ()))
    o_ref[...] = lax.dot_general(buf[slot], ident, dn,
                                 preferred_element_type=jnp.float32)


def _relayout_one(table):
    out = pl.pallas_call(
        _transpose_body,
        grid=(TR_GRID,),
        in_specs=[pl.BlockSpec(memory_space=pltpu.MemorySpace.HBM)],
        out_specs=pl.BlockSpec((TR_COLS, DIM), lambda i: (i, 0)),
        out_shape=jax.ShapeDtypeStruct((E_MAIN, DIM), jnp.float32),
        scratch_shapes=[
            pltpu.VMEM((2, DIM, TR_COLS), jnp.float32),
            pltpu.SemaphoreType.DMA((2, NQ)),
        ],
    )(table.T)
    return out.reshape(-1, PACK)


def _pack_idx(pos_triplets, neg_triplets):
    p = pos_triplets.astype(jnp.int32).reshape(BATCH // CHUNK, CHUNK, 3)
    n = neg_triplets.astype(jnp.int32).reshape(BATCH // CHUNK, CHUNK, 3)
    tri = jnp.concatenate([p.transpose(0, 2, 1), n.transpose(0, 2, 1)],
                          axis=1)
    is_rel = jnp.array([0, 1, 0, 0, 1, 0], jnp.bool_).reshape(1, 6, 1)
    pidx = jnp.where(is_rel, jnp.minimum(tri, E_MAIN - 1), tri) >> 1
    par = (tri & 1) * DIM
    trow = jnp.where(is_rel, (tri - E_MAIN) >> 1, -1)
    return pidx, par, trow


def kernel(pos_triplets, neg_triplets, e_table, r_table):
    pidx, par, trow = _pack_idx(pos_triplets, neg_triplets)
    e_packed = e_table[:-1].reshape(E_ROWS // 2, PACK)
    r_packed = _relayout_one(r_table)
    e_tail = e_table[E_MAIN:E_ROWS].reshape(N_TAIL // 2, PACK)
    r_tail = r_table[E_MAIN:E_ROWS].reshape(N_TAIL // 2, PACK)
    loss, pos_d, neg_d = _transe_sc(pidx, par, trow, e_packed, r_packed,
                                    e_tail, r_tail)
    return (loss, pos_d, neg_d)


# speedup vs baseline: 1.1638x; 1.1048x over previous
import functools

import jax
import jax.numpy as jnp
from jax import lax
from jax.experimental import pallas as pl
from jax.experimental.pallas import tpu as pltpu
from jax.experimental.pallas import tpu_sc as plsc

DIM = 64
BATCH = 16384
MARGIN = 1.0

NUM_CORES = 2
NUM_SUBCORES = 16
NUM_WORKERS = NUM_CORES * NUM_SUBCORES
ROWS_PER_W = BATCH // NUM_WORKERS
CHUNK = 128
CHUNKS_PER_HALF = ROWS_PER_W // CHUNK
GROUPS = CHUNK // 16


def _sc_body(idx_hbm, e_hbm, r_hbm, loss_hbm, pos_hbm, neg_hbm,
             idx_v, hbuf, rbuf, tbuf, dist_v, loss_v, sem):
    wid = lax.axis_index("s") * NUM_CORES + lax.axis_index("c")
    base = wid * ROWS_PER_W

    pltpu.sync_copy(idx_hbm.at[pl.ds(wid * CHUNKS_PER_HALF, CHUNKS_PER_HALF)],
                    idx_v)

    lane = lax.iota(jnp.int32, 16)

    def chunk_body(c, _):
        cc = c % CHUNKS_PER_HALF
        kind0 = (c // CHUNKS_PER_HALF) * 3

        dh = pltpu.async_copy(e_hbm.at[idx_v.at[cc, kind0]], hbuf, sem)
        dr = pltpu.async_copy(r_hbm.at[idx_v.at[cc, kind0 + 1]], rbuf, sem)
        dt = pltpu.async_copy(e_hbm.at[idx_v.at[cc, kind0 + 2]], tbuf, sem)
        dh.wait()
        dr.wait()
        dt.wait()

        def group_body(g, _):
            rows = g * 16 + lane
            nh = jnp.zeros((16,), jnp.float32)
            nt = jnp.zeros((16,), jnp.float32)
            for d in range(DIM):
                col = (lane + d) & (DIM - 1)
                nh = nh + jnp.abs(plsc.load_gather(hbuf, [rows, col]))
                nt = nt + jnp.abs(plsc.load_gather(tbuf, [rows, col]))
            inv_nh = 1.0 / nh
            inv_nt = 1.0 / nt
            acc = jnp.zeros((16,), jnp.float32)
            for d in range(DIM):
                col = (lane + d) & (DIM - 1)
                hv = plsc.load_gather(hbuf, [rows, col])
                rv = plsc.load_gather(rbuf, [rows, col])
                tv = plsc.load_gather(tbuf, [rows, col])
                acc = acc + jnp.abs(hv * inv_nh + rv - tv * inv_nt)
            dist_v[pl.ds(c * CHUNK + g * 16, 16)] = acc
            return 0

        lax.fori_loop(0, GROUPS, group_body, 0)
        return 0

    lax.fori_loop(0, 2 * CHUNKS_PER_HALF, chunk_body, 0)

    for v in range(ROWS_PER_W // 16):
        pv = dist_v[pl.ds(v * 16, 16)]
        nv = dist_v[pl.ds(ROWS_PER_W + v * 16, 16)]
        loss_v[pl.ds(v * 16, 16)] = jnp.maximum(pv - nv + MARGIN, 0.0)

    pltpu.sync_copy(loss_v, loss_hbm.at[pl.ds(base, ROWS_PER_W)])
    pltpu.sync_copy(dist_v.at[pl.ds(0, ROWS_PER_W)],
                    pos_hbm.at[pl.ds(base, ROWS_PER_W)])
    pltpu.sync_copy(dist_v.at[pl.ds(ROWS_PER_W, ROWS_PER_W)],
                    neg_hbm.at[pl.ds(base, ROWS_PER_W)])


@functools.partial(
    pl.kernel,
    out_type=(
        jax.ShapeDtypeStruct((BATCH,), jnp.float32),
        jax.ShapeDtypeStruct((BATCH,), jnp.float32),
        jax.ShapeDtypeStruct((BATCH,), jnp.float32),
    ),
    mesh=plsc.VectorSubcoreMesh(core_axis_name="c", subcore_axis_name="s",
                                num_cores=NUM_CORES,
                                num_subcores=NUM_SUBCORES),
    scratch_types=[
        pltpu.VMEM((CHUNKS_PER_HALF, 6, CHUNK), jnp.int32),
        pltpu.VMEM((CHUNK, DIM), jnp.float32),
        pltpu.VMEM((CHUNK, DIM), jnp.float32),
        pltpu.VMEM((CHUNK, DIM), jnp.float32),
        pltpu.VMEM((2 * ROWS_PER_W,), jnp.float32),
        pltpu.VMEM((ROWS_PER_W,), jnp.float32),
        pltpu.SemaphoreType.DMA,
    ],
    compiler_params=pltpu.CompilerParams(needs_layout_passes=False,
                                         use_tc_tiling_on_sc=False),
)
def _transe_sc(idx_hbm, e_hbm, r_hbm, loss_hbm, pos_hbm, neg_hbm,
               idx_v, hbuf, rbuf, tbuf, dist_v, loss_v, sem):
    _sc_body(idx_hbm, e_hbm, r_hbm, loss_hbm, pos_hbm, neg_hbm,
             idx_v, hbuf, rbuf, tbuf, dist_v, loss_v, sem)


def kernel(pos_triplets, neg_triplets, e_table, r_table):
    p = pos_triplets.astype(jnp.int32).reshape(BATCH // CHUNK, CHUNK, 3)
    n = neg_triplets.astype(jnp.int32).reshape(BATCH // CHUNK, CHUNK, 3)
    idx = jnp.concatenate([p.transpose(0, 2, 1), n.transpose(0, 2, 1)],
                          axis=1)
    loss, pos_d, neg_d = _transe_sc(idx, e_table, r_table)
    return (loss, pos_d, neg_d)
